# baseline (speedup 1.0000x reference)
import jax
import jax.numpy as jnp
from jax.experimental import pallas as pl
from jax.experimental.pallas import tpu as pltpu

_PACK = 8


def _packed_kernel(z_ref, w_ref, b_ref, o_ref):
    acc = jnp.dot(z_ref[...], w_ref[...], preferred_element_type=jnp.float32)
    o_ref[...] = (acc + b_ref[...]).astype(o_ref.dtype)


def kernel(z, w12, b_eff, w_bd, b_bd):
    B, in_dim = z.shape
    out_dim = w12.shape[1]

    if B % _PACK != 0:
        zp, w, b = z, w12, b_eff.reshape(1, out_dim)
        rows, k, n = B, in_dim, out_dim
    else:
        zp = z.reshape(B // _PACK, _PACK * in_dim)
        w, b = w_bd, b_bd
        rows, k, n = B // _PACK, _PACK * in_dim, _PACK * out_dim

    tile = 4096
    if rows % tile != 0:
        tile = 8 * max(1, rows // (8 * 8))
    if rows <= tile:
        out = pl.pallas_call(
            _packed_kernel,
            out_shape=jax.ShapeDtypeStruct((rows, n), z.dtype),
        )(zp, w, b)
    else:
        steps = pl.cdiv(rows, tile)
        out = pl.pallas_call(
            _packed_kernel,
            out_shape=jax.ShapeDtypeStruct((rows, n), z.dtype),
            grid=(steps,),
            in_specs=[
                pl.BlockSpec((tile, k), lambda i: (i, 0)),
                pl.BlockSpec((k, n), lambda i: (0, 0)),
                pl.BlockSpec((1, n), lambda i: (0, 0)),
            ],
            out_specs=pl.BlockSpec((tile, n), lambda i: (i, 0)),
            compiler_params=pltpu.CompilerParams(
                dimension_semantics=("parallel",),
                vmem_limit_bytes=60 * 1024 * 1024,
            ),
        )(zp, w, b)

    return out.reshape(B, out_dim)
